# baseline (speedup 1.0000x reference)
import functools
import jax
import jax.numpy as jnp
from jax import lax
from jax.experimental import pallas as pl
from jax.experimental.pallas import tpu as pltpu
from jax.experimental.pallas import tpu_sc as plsc

B = 4
N = 2048
K = 8
C = 512

_NC = 2
_NS = 16
_NW = _NC * _NS


def _leaky(x, a):
    return jnp.maximum(x, a * x)


def _knn_body(x_ref, xt_ref, sqc_ref, sqr_ref, idx_ref, *, n, k, inner_sq,
              ndim=0):
    b = pl.program_id(0)
    xr = x_ref[0]
    xt = xt_ref[0]
    if inner_sq:
        sq_all = jnp.sum(xt * xt, axis=0, keepdims=True)
    else:
        sq_all = sqr_ref[0]
    sq_r = sqc_ref[0]
    if ndim:
        g = xr[:, 0:1] * xt[0:1, :]
        for c in range(1, ndim):
            g = g + xr[:, c:c + 1] * xt[c:c + 1, :]
    else:
        g = jnp.dot(xr, xt, preferred_element_type=jnp.float32)
    d = sq_r + sq_all - 2.0 * g
    neg = -d
    r = xr.shape[0]
    iota = jax.lax.broadcasted_iota(jnp.int32, (r, n), 1)
    cols = []
    for _ in range(k):
        m = jnp.max(neg, axis=1, keepdims=True)
        cand = jnp.where(neg == m, iota, n)
        sel = jnp.min(cand, axis=1, keepdims=True)
        cols.append(sel)
        neg = jnp.where(iota == sel, -jnp.inf, neg)
    idx_ref[0] = jnp.concatenate(cols, axis=1) + b * n


def _knn(x, sq=None, ndim=0):
    b, n, c = x.shape
    xt = jnp.swapaxes(x, 1, 2)
    inner_sq = sq is None
    if sq is None:
        sq = jnp.sum(xt * xt, axis=1)
    sq_col = sq[:, :, None]
    sq_row = sq[:, None, :]
    r = 256
    return pl.pallas_call(
        functools.partial(_knn_body, n=n, k=K, inner_sq=inner_sq, ndim=ndim),
        grid=(b, n // r),
        in_specs=[
            pl.BlockSpec((1, r, c), lambda bi, i: (bi, i, 0)),
            pl.BlockSpec((1, c, n), lambda bi, i: (bi, 0, 0)),
            pl.BlockSpec((1, r, 1), lambda bi, i: (bi, i, 0)),
            pl.BlockSpec((1, 1, n), lambda bi, i: (bi, 0, 0)),
        ],
        out_specs=pl.BlockSpec((1, r, K), lambda bi, i: (bi, i, 0)),
        out_shape=jax.ShapeDtypeStruct((b, n, K), jnp.int32),
    )(x, xt, sq_col, sq_row)


def _linear_body(x_ref, wt_ref, b_ref, o_ref, *, act):
    t = jnp.dot(x_ref[0], wt_ref[...], preferred_element_type=jnp.float32)
    t = t + b_ref[...]
    if act == "leaky":
        t = _leaky(t, 0.2)
    o_ref[0] = t


def _linear(x, wt, bias, act):
    b, n, c = x.shape
    o = wt.shape[1]
    r = 512
    return pl.pallas_call(
        functools.partial(_linear_body, act=act),
        grid=(b, n // r),
        in_specs=[
            pl.BlockSpec((1, r, c), lambda bi, i: (bi, i, 0)),
            pl.BlockSpec((c, o), lambda bi, i: (0, 0)),
            pl.BlockSpec((1, o), lambda bi, i: (0, 0)),
        ],
        out_specs=pl.BlockSpec((1, r, o), lambda bi, i: (bi, i, 0)),
        out_shape=jax.ShapeDtypeStruct((b, n, o), jnp.float32),
    )(x, wt, bias)


def _sc_gmean_kernel(affine):
    npts = B * N
    ppw = npts // _NW
    cpk = 16
    rows = cpk * K
    nch = ppw // cpk
    mesh = plsc.VectorSubcoreMesh(core_axis_name="c", subcore_axis_name="s")

    def body(tab_hbm, idx_hbm, *rest):
        if affine:
            ac_hbm, out_hbm, idx_v, rows_v, out_v, ac_v, sem = rest
        else:
            out_hbm, idx_v, rows_v, out_v, sem = rest
        wid = lax.axis_index("s") * _NC + lax.axis_index("c")
        pltpu.sync_copy(idx_hbm.at[pl.ds(wid * nch, nch)], idx_v)
        if affine:
            pltpu.sync_copy(ac_hbm, ac_v)

        def chunk(ch, _):
            pltpu.async_copy(tab_hbm.at[idx_v.at[ch]], rows_v, sem).wait()

            def point(p, _):
                r0 = p * K
                for c in range(C // 16):
                    sl = pl.ds(c * 16, 16)
                    h = [rows_v[r0 + kk, sl] for kk in range(K)]
                    if affine:
                        a = ac_v[0, sl]
                        cc = ac_v[1, sl]
                        h = [_leaky(a * t + cc, 0.2) for t in h]
                    out_v[p, sl] = (((h[0] + h[4]) + (h[2] + h[6]))
                                    + ((h[1] + h[5]) + (h[3] + h[7]))) * 0.125
                return 0

            lax.fori_loop(0, cpk, point, 0)
            pltpu.sync_copy(
                out_v, out_hbm.at[pl.ds(wid * ppw + ch * cpk, cpk)])
            return 0

        lax.fori_loop(0, nch, chunk, 0)

    scratch = [
        pltpu.VMEM((nch, rows), jnp.int32),
        pltpu.VMEM((rows, C), jnp.float32),
        pltpu.VMEM((cpk, C), jnp.float32),
    ]
    if affine:
        scratch.append(pltpu.VMEM((2, C), jnp.float32))
    scratch.append(pltpu.SemaphoreType.DMA)

    return pl.kernel(
        body,
        out_type=jax.ShapeDtypeStruct((npts, C), jnp.float32),
        mesh=mesh,
        scratch_types=scratch,
    )


def _sc_gather_mean(table, idx, ac=None):
    tab = table.reshape(B * N, C)
    idx2d = idx.reshape(-1, 16 * K)
    if ac is None:
        out = _sc_gmean_kernel(False)(tab, idx2d)
    else:
        out = _sc_gmean_kernel(True)(tab, idx2d, ac)
    return out.reshape(B, N, C)


def _tree_mean8(t):
    return (((t[0] + t[4]) + (t[2] + t[6]))
            + ((t[1] + t[5]) + (t[3] + t[7]))) * 0.125


def _layer1_body(idxt_ref, st_ref, w1p_ref, zt_ref, b1c_ref, o_ref, *, n):
    i = pl.program_id(0)
    r = o_ref.shape[2]
    idxt = idxt_ref[...]
    st = st_ref[...]
    s_blkt = st_ref[:, pl.ds(i * r, r)]
    w1p = w1p_ref[...]
    b1c = b1c_ref[...]
    iota = jax.lax.broadcasted_iota(jnp.int32, (n, r), 0)
    parts = []
    for k in range(K):
        mkt = (iota == idxt[k:k + 1, :]).astype(jnp.float32)
        gxt = jnp.dot(st, mkt, preferred_element_type=jnp.float32,
                      precision=jax.lax.Precision.HIGHEST)
        dt = gxt - s_blkt
        parts.append((s_blkt[0:3, :], dt[0:3, :]))
    for bi in range(B):
        zcol = jnp.broadcast_to(zt_ref[:, bi:bi + 1], (C, r))
        ts = []
        for k in range(K):
            sx3, d3 = parts[k]
            f1 = jnp.concatenate(
                [sx3, d3, zcol, jnp.zeros((2, r), jnp.float32)], axis=0)
            y = jnp.dot(w1p, f1, preferred_element_type=jnp.float32)
            ts.append(_leaky(y + b1c, 0.2))
        o_ref[bi] = _tree_mean8(ts)


def _layer1(idx0, s_pad, w1, z, b1):
    n = s_pad.shape[0]
    idxt = idx0.T
    st = s_pad.T
    w1p = jnp.pad(w1, ((0, 0), (0, 2)))
    zt = z.T
    b1c = b1[:, None]
    r = 256
    return pl.pallas_call(
        functools.partial(_layer1_body, n=n),
        grid=(n // r,),
        in_specs=[
            pl.BlockSpec((K, r), lambda i: (0, i)),
            pl.BlockSpec((8, n), lambda i: (0, 0)),
            pl.BlockSpec((C, 520), lambda i: (0, 0)),
            pl.BlockSpec((C, B), lambda i: (0, 0)),
            pl.BlockSpec((C, 1), lambda i: (0, 0)),
        ],
        out_specs=pl.BlockSpec((B, C, r), lambda i: (0, 0, i)),
        out_shape=jax.ShapeDtypeStruct((B, C, n), jnp.float32),
    )(idxt, st, w1p, zt, b1c)


def _stats_body(idx_ref, q_ref, gm_ref, bt_ref, ac_ref, s_ref, *, n):
    b = pl.program_id(0)
    idx = idx_ref[0]
    q = q_ref[0]
    cnt = jnp.zeros((1, n), jnp.float32)
    r = 256
    for i in range(n // r):
        blk = idx[i * r:(i + 1) * r]
        iota = jax.lax.broadcasted_iota(jnp.int32, (r, n), 1) + b * n
        m = jnp.zeros((r, n), jnp.float32)
        for k in range(K):
            m = m + (iota == blk[:, k:k + 1]).astype(jnp.float32)
        cnt = cnt + jnp.sum(m, axis=0, keepdims=True)
    s1 = jnp.dot(cnt, q, preferred_element_type=jnp.float32,
                 precision=jax.lax.Precision.HIGHEST)
    s2 = jnp.dot(cnt, q * q, preferred_element_type=jnp.float32,
                 precision=jax.lax.Precision.HIGHEST)

    @pl.when(b == 0)
    def _():
        s_ref[...] = jnp.zeros_like(s_ref)

    s_ref[0:1, :] += s1
    s_ref[1:2, :] += s2

    @pl.when(b == pl.num_programs(0) - 1)
    def _():
        denom = 1.0 / (B * n * K)
        mu = s_ref[0:1, :] * denom
        var = s_ref[1:2, :] * denom - mu * mu
        a = gm_ref[...] / jnp.sqrt(var + 1e-5)
        c = bt_ref[...] - mu * a
        ac_ref[...] = jnp.concatenate([a, c], axis=0)


def _stats(idx4, q, gamma, beta):
    b, n, c = q.shape
    return pl.pallas_call(
        functools.partial(_stats_body, n=n),
        grid=(b,),
        in_specs=[
            pl.BlockSpec((1, n, K), lambda bi: (bi, 0, 0)),
            pl.BlockSpec((1, n, c), lambda bi: (bi, 0, 0)),
            pl.BlockSpec((1, c), lambda bi: (0, 0)),
            pl.BlockSpec((1, c), lambda bi: (0, 0)),
        ],
        out_specs=pl.BlockSpec((2, c), lambda bi: (0, 0)),
        out_shape=jax.ShapeDtypeStruct((2, c), jnp.float32),
        scratch_shapes=[pltpu.VMEM((2, c), jnp.float32)],
    )(idx4, q, gamma, beta)


def _head_body(f_ref, w1_ref, b1_ref, w2_ref, b2_ref, o_ref):
    t = jnp.dot(f_ref[0], w1_ref[...], preferred_element_type=jnp.float32)
    t = _leaky(t + b1_ref[...], 0.01)
    u = jnp.dot(t, w2_ref[...], preferred_element_type=jnp.float32)
    u = u + b2_ref[...]
    o_ref[0] = 1.0 / (1.0 + jnp.exp(-u)) - 0.5


def _head(feat, w1t, b1, w2t, b2):
    b, n, c = feat.shape
    o = w2t.shape[1]
    return pl.pallas_call(
        _head_body,
        grid=(b,),
        in_specs=[
            pl.BlockSpec((1, n, c), lambda bi: (bi, 0, 0)),
            pl.BlockSpec((c, 64), lambda bi: (0, 0)),
            pl.BlockSpec((1, 64), lambda bi: (0, 0)),
            pl.BlockSpec((64, o), lambda bi: (0, 0)),
            pl.BlockSpec((1, o), lambda bi: (0, 0)),
        ],
        out_specs=pl.BlockSpec((1, n, o), lambda bi: (bi, 0, 0)),
        out_shape=jax.ShapeDtypeStruct((b, n, o), jnp.float32),
    )(feat, w1t, b1, w2t, b2)


def kernel(z, point_num, sphere, W1, b1, W2, b2, W3, b3, W4, b4, gamma, beta,
           Wx1, bx1, Wx2, bx2):
    n = sphere.shape[1]
    s = sphere[0]
    s_pad = jnp.pad(s, ((0, 0), (0, 5)))

    idx0 = _knn(s_pad[None], jnp.sum(sphere * sphere, axis=-1))
    out1_cn = _layer1(idx0[0], s_pad, W1, z, b1)
    out1 = jnp.swapaxes(out1_cn, 1, 2)

    idx2 = _knn(out1)
    p2 = _linear(out1, W2.T, b2[None, :], "leaky")
    out2 = _sc_gather_mean(p2, idx2)

    idx3 = _knn(out2)
    p3 = _linear(out2, W3.T, b3[None, :], "leaky")
    out3 = _sc_gather_mean(p3, idx3)

    idx4 = _knn(out3)
    q = _linear(out3, W4.T, b4[None, :], "none")
    ac = _stats(idx4, q, gamma[None, :], beta[None, :])
    feat_nt = _sc_gather_mean(q, idx4, ac)

    head_out = _head(feat_nt, Wx1.T, bx1[None, :],
                     jnp.pad(Wx2.T, ((0, 0), (0, 5))),
                     jnp.pad(bx2, (0, 5))[None, :])
    pcs = head_out[:, :, :3]
    feature = jnp.swapaxes(feat_nt, 1, 2)
    return pcs, feature
